# baseline (speedup 1.0000x reference)
import functools

import jax
import jax.numpy as jnp
from jax import lax
from jax.experimental import pallas as pl
from jax.experimental.pallas import tpu as pltpu
from jax.experimental.pallas import tpu_sc as plsc

N = 10000
E = 320000
H = 128
NC = 2
NS = 16
NW = NC * NS
CHUNK = 64
EPW = 10240
E2 = EPW * NW
NCHUNK = EPW // CHUNK
IBLK = 32
NIB = NCHUNK // IBLK
AGGR = N
RCL = 40
RPT = 624
REM = N - RPT * NS

_DN = (((1,), (0,)), ((), ()))


_BE = 2560


def _edge_mlp_body(ea_ref, es_ref, w1a_ref, w1s_ref, b1_ref, w2_ref, b2_ref,
                   o_ref):
    ta = lax.dot_general(ea_ref[...], w1a_ref[...], _DN,
                         preferred_element_type=jnp.float32)
    ts = lax.dot_general(es_ref[...], w1s_ref[...], _DN,
                         preferred_element_type=jnp.float32)
    t = ta.reshape(_BE, H) + ts.reshape(_BE, H) + b1_ref[...]
    t = t * jax.nn.sigmoid(t)
    u = lax.dot_general(t, w2_ref[...], _DN,
                        preferred_element_type=jnp.float32)
    o_ref[...] = u + b2_ref[...]


def _edge_mlp(edge_attr, edge_sh, W1e, b1e, W2e, b2e):
    BE = _BE
    grid = (E2 // BE,)
    last = E // BE - 1
    ea8 = edge_attr.reshape(E // 8, 128)
    es32 = edge_sh.reshape(E // 32, 128)
    w1a_bd = jnp.kron(jnp.eye(8, dtype=jnp.float32), W1e[:16])
    w1s_bd = jnp.kron(jnp.eye(32, dtype=jnp.float32), W1e[16:20])
    return pl.pallas_call(
        _edge_mlp_body,
        grid=grid,
        in_specs=[
            pl.BlockSpec((BE // 8, 128), lambda i: (jnp.minimum(i, last), 0)),
            pl.BlockSpec((BE // 32, 128), lambda i: (jnp.minimum(i, last), 0)),
            pl.BlockSpec((128, 8 * H), lambda i: (0, 0)),
            pl.BlockSpec((128, 32 * H), lambda i: (0, 0)),
            pl.BlockSpec((1, H), lambda i: (0, 0)),
            pl.BlockSpec((H, H), lambda i: (0, 0)),
            pl.BlockSpec((1, H), lambda i: (0, 0)),
        ],
        out_specs=pl.BlockSpec((BE, H), lambda i: (i, 0)),
        out_shape=jax.ShapeDtypeStruct((E2, H), jnp.float32),
    )(ea8, es32, w1a_bd, w1s_bd, b1e.reshape(1, H), W2e, b2e.reshape(1, H))


def _sc_messages(h, src_blk, dst_blk, w_edge):
    mesh = plsc.VectorSubcoreMesh(core_axis_name="core", subcore_axis_name="subcore")

    @functools.partial(
        pl.kernel,
        out_type=jax.ShapeDtypeStruct((NC, N, H), jnp.float32),
        mesh=mesh,
        scratch_types=[
            pltpu.VMEM((IBLK, CHUNK), jnp.int32),
            pltpu.VMEM((IBLK, CHUNK), jnp.int32),
            pltpu.VMEM((CHUNK, H), jnp.float32),
            pltpu.VMEM((CHUNK, H), jnp.float32),
            pltpu.VMEM((CHUNK, H), jnp.float32),
            pltpu.VMEM((CHUNK, H), jnp.float32),
            pltpu.VMEM_SHARED((AGGR, H), jnp.float32),
            pltpu.SemaphoreType.DMA,
            pltpu.SemaphoreType.DMA,
            pltpu.SemaphoreType.DMA,
            pltpu.SemaphoreType.DMA,
        ],
    )
    def k(h_hbm, src_hbm, dst_hbm, we_hbm, out_hbm,
          src_v, dst_v, rows0, rows1, wv0, wv1, agg, sg0, sg1, sw0, sw1):
        cid = lax.axis_index("core")
        sid = lax.axis_index("subcore")
        wid = cid * NS + sid
        rows = (rows0, rows1)
        wv = (wv0, wv1)
        sg = (sg0, sg1)
        sw = (sw0, sw1)

        @pl.loop(0, CHUNK)
        def _(i):
            for kk in range(H // 16):
                rows0[i, pl.ds(kk * 16, 16)] = jnp.zeros((16,), jnp.float32)

        zbase = pl.multiple_of(sid * RPT, 8)
        for r in range(9):
            pltpu.sync_copy(rows0, agg.at[pl.ds(zbase + r * CHUNK, CHUNK)])
        pltpu.sync_copy(rows0.at[pl.ds(0, RPT - 9 * CHUNK)],
                        agg.at[pl.ds(zbase + 9 * CHUNK, RPT - 9 * CHUNK)])

        @pl.when(sid == NS - 1)
        def _():
            pltpu.sync_copy(rows0.at[pl.ds(0, REM)],
                            agg.at[pl.ds(NS * RPT, REM)])

        plsc.subcore_barrier()

        def issue(j, buf, base):
            cg = pltpu.async_copy(h_hbm.at[src_v.at[j]], rows[buf], sg[buf])
            cw = pltpu.async_copy(we_hbm.at[pl.ds(base + j * CHUNK, CHUNK)],
                                  wv[buf], sw[buf])
            return cg, cw

        def process(j, buf, base, chunk0):
            pltpu.make_async_copy(h_hbm.at[src_v.at[j]], rows[buf],
                                  sg[buf]).wait()
            pltpu.make_async_copy(we_hbm.at[pl.ds(base + j * CHUNK, CHUNK)],
                                  wv[buf], sw[buf]).wait()

            @pl.loop(0, CHUNK)
            def _(e):
                for kk in range(H // 16):
                    s = pl.ds(kk * 16, 16)
                    rows[buf][e, s] = rows[buf][e, s] * wv[buf][e, s]

            @pl.when(jnp.logical_or(wid != NW - 1, chunk0 + j < RCL))
            def _():
                pltpu.sync_copy(rows[buf], agg.at[dst_v.at[j]], add=True)

        for b in range(NIB):
            pltpu.sync_copy(src_hbm.at[wid, b], src_v)
            pltpu.sync_copy(dst_hbm.at[wid, b], dst_v)
            base = wid * EPW + b * IBLK * CHUNK

            issue(0, 0, base)

            @pl.loop(0, IBLK // 2)
            def _(p):
                issue(2 * p + 1, 1, base)
                process(2 * p, 0, base, b * IBLK)

                @pl.when(p + 1 < IBLK // 2)
                def _():
                    issue(2 * p + 2, 0, base)

                process(2 * p + 1, 1, base, b * IBLK)

        plsc.subcore_barrier()
        sl = pl.ds(zbase, RPT)
        pltpu.sync_copy(agg.at[sl], out_hbm.at[cid, sl])

        @pl.when(sid == NS - 1)
        def _():
            slr = pl.ds(NS * RPT, REM)
            pltpu.sync_copy(agg.at[slr], out_hbm.at[cid, slr])

    return k(h, src_blk, dst_blk, w_edge)


def _node_body(h_ref, p_ref, w1_ref, b1_ref, w2_ref, b2_ref,
               g_ref, bb_ref, o_ref):
    hb = h_ref[...]
    agg = p_ref[0] + p_ref[1]
    t = lax.dot_general(hb, w1_ref[:H], _DN,
                        preferred_element_type=jnp.float32)
    t = t + lax.dot_general(agg, w1_ref[H:], _DN,
                            preferred_element_type=jnp.float32)
    t = t + b1_ref[...]
    t = t * jax.nn.sigmoid(t)
    u = lax.dot_general(t, w2_ref[...], _DN,
                        preferred_element_type=jnp.float32)
    x = hb + u + b2_ref[...]
    mean = jnp.mean(x, axis=1, keepdims=True)
    var = jnp.mean((x - mean) ** 2, axis=1, keepdims=True)
    o_ref[...] = (x - mean) / jnp.sqrt(var + 1e-5) * g_ref[...] + bb_ref[...]


def _node_update(h, partials, W1n, b1n, W2n, b2n, gamma, beta):
    BN = 2000
    grid = (N // BN,)
    return pl.pallas_call(
        _node_body,
        grid=grid,
        in_specs=[
            pl.BlockSpec((BN, H), lambda i: (i, 0)),
            pl.BlockSpec((NC, BN, H), lambda i: (0, i, 0)),
            pl.BlockSpec((2 * H, H), lambda i: (0, 0)),
            pl.BlockSpec((1, H), lambda i: (0, 0)),
            pl.BlockSpec((H, H), lambda i: (0, 0)),
            pl.BlockSpec((1, H), lambda i: (0, 0)),
            pl.BlockSpec((1, H), lambda i: (0, 0)),
            pl.BlockSpec((1, H), lambda i: (0, 0)),
        ],
        out_specs=pl.BlockSpec((BN, H), lambda i: (i, 0)),
        out_shape=jax.ShapeDtypeStruct((N, H), jnp.float32),
    )(h, partials, W1n, b1n.reshape(1, H), W2n, b2n.reshape(1, H),
      gamma.reshape(1, H), beta.reshape(1, H))


def kernel(h, edge_index, edge_attr, edge_sh,
           W1e, b1e, W2e, b2e, W1n, b1n, W2n, b2n, gamma, beta):
    pad = E2 - E
    src_blk = jnp.concatenate(
        [edge_index[0], jnp.arange(pad, dtype=jnp.int32) % N]
    ).reshape(NW, NIB, IBLK, CHUNK)
    dst_blk = jnp.concatenate(
        [edge_index[1], jnp.zeros((pad,), jnp.int32)]
    ).reshape(NW, NIB, IBLK, CHUNK)

    w_edge = _edge_mlp(edge_attr, edge_sh, W1e, b1e, W2e, b2e)
    partials = _sc_messages(h, src_blk, dst_blk, w_edge)
    return _node_update(h, partials, W1n, b1n, W2n, b2n, gamma, beta)


# speedup vs baseline: 1.3973x; 1.3973x over previous
import functools

import jax
import jax.numpy as jnp
from jax import lax
from jax.experimental import pallas as pl
from jax.experimental.pallas import tpu as pltpu
from jax.experimental.pallas import tpu_sc as plsc

N = 10000
E = 320000
H = 128
NC = 2
NS = 16
NW = NC * NS
CHUNK = 64
EPW = 10240
E2 = EPW * NW
NCHUNK = EPW // CHUNK
IBLK = 32
NIB = NCHUNK // IBLK
AGGR = N
RCL = 40
RPT = 624
REM = N - RPT * NS

_DN = (((1,), (0,)), ((), ()))


def _edge_mlp_body(x_ref, w1_ref, b1_ref, w2_ref, b2_ref, o_ref):
    t = lax.dot_general(x_ref[...], w1_ref[...], _DN,
                        preferred_element_type=jnp.float32)
    t = t + b1_ref[...]
    t = t * jax.nn.sigmoid(t)
    u = lax.dot_general(t, w2_ref[...], _DN,
                        preferred_element_type=jnp.float32)
    o_ref[...] = u + b2_ref[...]


def _edge_mlp(edge_in, W1e, b1e, W2e, b2e):
    BE = 2560
    grid = (E2 // BE,)
    last = E // BE - 1
    return pl.pallas_call(
        _edge_mlp_body,
        grid=grid,
        in_specs=[
            pl.BlockSpec((BE, 20), lambda i: (jnp.minimum(i, last), 0)),
            pl.BlockSpec(W1e.shape, lambda i: (0, 0)),
            pl.BlockSpec((1, H), lambda i: (0, 0)),
            pl.BlockSpec((H, H), lambda i: (0, 0)),
            pl.BlockSpec((1, H), lambda i: (0, 0)),
        ],
        out_specs=pl.BlockSpec((BE, H), lambda i: (i, 0)),
        out_shape=jax.ShapeDtypeStruct((E2, H), jnp.float32),
    )(edge_in, W1e, b1e.reshape(1, H), W2e, b2e.reshape(1, H))


def _sc_messages(h, src_blk, dst_blk, w_edge):
    mesh = plsc.VectorSubcoreMesh(core_axis_name="core", subcore_axis_name="subcore")

    @functools.partial(
        pl.kernel,
        out_type=jax.ShapeDtypeStruct((NC, N, H), jnp.float32),
        mesh=mesh,
        scratch_types=[
            pltpu.VMEM((IBLK, CHUNK), jnp.int32),
            pltpu.VMEM((IBLK, CHUNK), jnp.int32),
            pltpu.VMEM((CHUNK, H), jnp.float32),
            pltpu.VMEM((CHUNK, H), jnp.float32),
            pltpu.VMEM((CHUNK, H), jnp.float32),
            pltpu.VMEM((CHUNK, H), jnp.float32),
            pltpu.VMEM_SHARED((AGGR, H), jnp.float32),
            pltpu.SemaphoreType.DMA,
            pltpu.SemaphoreType.DMA,
            pltpu.SemaphoreType.DMA,
            pltpu.SemaphoreType.DMA,
        ],
    )
    def k(h_hbm, src_hbm, dst_hbm, we_hbm, out_hbm,
          src_v, dst_v, rows0, rows1, wv0, wv1, agg, sg0, sg1, sw0, sw1):
        cid = lax.axis_index("core")
        sid = lax.axis_index("subcore")
        wid = cid * NS + sid
        rows = (rows0, rows1)
        wv = (wv0, wv1)
        sg = (sg0, sg1)
        sw = (sw0, sw1)

        @pl.loop(0, CHUNK)
        def _(i):
            for kk in range(H // 16):
                rows0[i, pl.ds(kk * 16, 16)] = jnp.zeros((16,), jnp.float32)

        zbase = pl.multiple_of(sid * RPT, 8)
        for r in range(9):
            pltpu.sync_copy(rows0, agg.at[pl.ds(zbase + r * CHUNK, CHUNK)])
        pltpu.sync_copy(rows0.at[pl.ds(0, RPT - 9 * CHUNK)],
                        agg.at[pl.ds(zbase + 9 * CHUNK, RPT - 9 * CHUNK)])

        @pl.when(sid == NS - 1)
        def _():
            pltpu.sync_copy(rows0.at[pl.ds(0, REM)],
                            agg.at[pl.ds(NS * RPT, REM)])

        plsc.subcore_barrier()

        def issue(j, buf, base):
            cg = pltpu.async_copy(h_hbm.at[src_v.at[j]], rows[buf], sg[buf])
            cw = pltpu.async_copy(we_hbm.at[pl.ds(base + j * CHUNK, CHUNK)],
                                  wv[buf], sw[buf])
            return cg, cw

        def process(j, buf, base, chunk0):
            pltpu.make_async_copy(h_hbm.at[src_v.at[j]], rows[buf],
                                  sg[buf]).wait()
            pltpu.make_async_copy(we_hbm.at[pl.ds(base + j * CHUNK, CHUNK)],
                                  wv[buf], sw[buf]).wait()

            @pl.loop(0, CHUNK)
            def _(e):
                for kk in range(H // 16):
                    s = pl.ds(kk * 16, 16)
                    rows[buf][e, s] = rows[buf][e, s] * wv[buf][e, s]

            @pl.when(jnp.logical_or(wid != NW - 1, chunk0 + j < RCL))
            def _():
                pltpu.sync_copy(rows[buf], agg.at[dst_v.at[j]], add=True)

        for b in range(NIB):
            pltpu.sync_copy(src_hbm.at[wid, b], src_v)
            pltpu.sync_copy(dst_hbm.at[wid, b], dst_v)
            base = wid * EPW + b * IBLK * CHUNK

            issue(0, 0, base)

            @pl.loop(0, IBLK // 2)
            def _(p):
                issue(2 * p + 1, 1, base)
                process(2 * p, 0, base, b * IBLK)

                @pl.when(p + 1 < IBLK // 2)
                def _():
                    issue(2 * p + 2, 0, base)

                process(2 * p + 1, 1, base, b * IBLK)

        plsc.subcore_barrier()
        sl = pl.ds(zbase, RPT)
        pltpu.sync_copy(agg.at[sl], out_hbm.at[cid, sl])

        @pl.when(sid == NS - 1)
        def _():
            slr = pl.ds(NS * RPT, REM)
            pltpu.sync_copy(agg.at[slr], out_hbm.at[cid, slr])

    return k(h, src_blk, dst_blk, w_edge)


def _node_body(h_ref, p_ref, w1_ref, b1_ref, w2_ref, b2_ref,
               g_ref, bb_ref, o_ref):
    hb = h_ref[...]
    agg = p_ref[0] + p_ref[1]
    t = lax.dot_general(hb, w1_ref[:H], _DN,
                        preferred_element_type=jnp.float32)
    t = t + lax.dot_general(agg, w1_ref[H:], _DN,
                            preferred_element_type=jnp.float32)
    t = t + b1_ref[...]
    t = t * jax.nn.sigmoid(t)
    u = lax.dot_general(t, w2_ref[...], _DN,
                        preferred_element_type=jnp.float32)
    x = hb + u + b2_ref[...]
    mean = jnp.mean(x, axis=1, keepdims=True)
    var = jnp.mean((x - mean) ** 2, axis=1, keepdims=True)
    o_ref[...] = (x - mean) / jnp.sqrt(var + 1e-5) * g_ref[...] + bb_ref[...]


def _node_update(h, partials, W1n, b1n, W2n, b2n, gamma, beta):
    BN = 2000
    grid = (N // BN,)
    return pl.pallas_call(
        _node_body,
        grid=grid,
        in_specs=[
            pl.BlockSpec((BN, H), lambda i: (i, 0)),
            pl.BlockSpec((NC, BN, H), lambda i: (0, i, 0)),
            pl.BlockSpec((2 * H, H), lambda i: (0, 0)),
            pl.BlockSpec((1, H), lambda i: (0, 0)),
            pl.BlockSpec((H, H), lambda i: (0, 0)),
            pl.BlockSpec((1, H), lambda i: (0, 0)),
            pl.BlockSpec((1, H), lambda i: (0, 0)),
            pl.BlockSpec((1, H), lambda i: (0, 0)),
        ],
        out_specs=pl.BlockSpec((BN, H), lambda i: (i, 0)),
        out_shape=jax.ShapeDtypeStruct((N, H), jnp.float32),
    )(h, partials, W1n, b1n.reshape(1, H), W2n, b2n.reshape(1, H),
      gamma.reshape(1, H), beta.reshape(1, H))


def kernel(h, edge_index, edge_attr, edge_sh,
           W1e, b1e, W2e, b2e, W1n, b1n, W2n, b2n, gamma, beta):
    pad = E2 - E
    src_blk = jnp.concatenate(
        [edge_index[0], jnp.arange(pad, dtype=jnp.int32) % N]
    ).reshape(NW, NIB, IBLK, CHUNK)
    dst_blk = jnp.concatenate(
        [edge_index[1], jnp.zeros((pad,), jnp.int32)]
    ).reshape(NW, NIB, IBLK, CHUNK)

    edge_in = jnp.concatenate([edge_attr, edge_sh], axis=1)
    w_edge = _edge_mlp(edge_in, W1e, b1e, W2e, b2e)
    partials = _sc_messages(h, src_blk, dst_blk, w_edge)
    return _node_update(h, partials, W1n, b1n, W2n, b2n, gamma, beta)
